# Initial kernel scaffold; baseline (speedup 1.0000x reference)
import jax
import jax.numpy as jnp
from jax.experimental import pallas as pl


def kernel(adjacency_matrices, in_gamma, in_beta, W0, b0, bn_g0, bn_b0, W1, b1, bn_g1, bn_b1, W2, b2, bn_g2, bn_b2, ln_g, ln_b):
    raise NotImplementedError("write your pallas kernel here")


# speedup vs baseline: 1.5703x; 1.5703x over previous
import jax
import jax.numpy as jnp
from jax.experimental import pallas as pl

N_Q, N_X, N_Z = 80, 24, 24
N = N_Q + N_X + N_Z
B = 256
HID = 256
EPS = 1e-5
G = 16
STEPS = B // G
NROWS = B * N

_BF16 = jnp.bfloat16


def _adj_prep(adj_blk):
    a = (adj_blk != 0).astype(jnp.float32)
    row = jax.lax.broadcasted_iota(jnp.int32, (N, N), 0)
    col = jax.lax.broadcasted_iota(jnp.int32, (N, N), 1)
    a_hat = jnp.where((row == col)[None, :, :], 1.0, a)
    deg = jnp.sum(a_hat, axis=1)
    dinv = 1.0 / jnp.sqrt(deg)
    return a_hat, dinv


def _aggregate(a_hat, dinv, lin, bias):
    msg = (dinv[:, :, None] * lin).astype(_BF16)
    agg = jax.lax.dot_general(
        a_hat.astype(_BF16), msg, (((1,), (1,)), ((0,), (0,))),
        preferred_element_type=jnp.float32)
    return agg * dinv[:, :, None] + bias[None]


def _acc_stats(step, agg, stats_out):
    s = jnp.sum(agg, axis=(0, 1))
    q = jnp.sum(agg * agg, axis=(0, 1))
    @pl.when(step == 0)
    def _():
        stats_out[...] = jnp.zeros_like(stats_out)
    stats_out[0:2, :] += jnp.stack([s, q], axis=0)


def _bn(agg, stats, g, bt):
    m = stats[0:1, :] * (1.0 / NROWS)
    q = stats[1:2, :] * (1.0 / NROWS)
    inv = jax.lax.rsqrt((q - m * m) + EPS)
    return (agg - m[None]) * (inv * g)[None] + bt[None]


def _stage1_kernel(adj_ref, gam_ref, bet_ref, w0_ref, b0_ref, agg_out, stats_out):
    step = pl.program_id(0)
    ridx = jax.lax.broadcasted_iota(jnp.int32, (N, 1), 0)
    group = jnp.where(ridx < N_Q, 0, jnp.where(ridx < N_Q + N_X, 1, 2))
    cidx = jax.lax.broadcasted_iota(jnp.int32, (N, 3), 1)
    x = (cidx == group).astype(jnp.float32)
    c3 = jax.lax.broadcasted_iota(jnp.int32, (1, 3), 1)
    p = jnp.where(c3 == 0, N_Q / N,
                  jnp.where(c3 == 1, N_X / N, N_Z / N)).astype(jnp.float32)
    xf = (x - p) / jnp.sqrt(p * (1.0 - p) + EPS) * gam_ref[...] + bet_ref[...]
    lin = jax.lax.dot_general(
        xf.astype(_BF16), w0_ref[...].astype(_BF16), (((1,), (0,)), ((), ())),
        preferred_element_type=jnp.float32)
    a_hat, dinv = _adj_prep(adj_ref[...])
    lin_b = jnp.broadcast_to(lin[None], (G, N, HID))
    agg = _aggregate(a_hat, dinv, lin_b, b0_ref[...])
    agg_out[...] = agg
    _acc_stats(step, agg, stats_out)


def _mid_kernel_res(agg_ref, stats_ref, g_ref, bt_ref, res_ref, adj_ref,
                    w_ref, b_ref, h_out, agg_out, stats_out):
    _mid_body(True, agg_ref, stats_ref, g_ref, bt_ref, res_ref, adj_ref,
              w_ref, b_ref, h_out, agg_out, stats_out)


def _mid_kernel_nores(agg_ref, stats_ref, g_ref, bt_ref, adj_ref,
                      w_ref, b_ref, h_out, agg_out, stats_out):
    _mid_body(False, agg_ref, stats_ref, g_ref, bt_ref, None, adj_ref,
              w_ref, b_ref, h_out, agg_out, stats_out)


def _mid_body(has_res, agg_ref, stats_ref, g_ref, bt_ref, res_ref, adj_ref,
              w_ref, b_ref, h_out, agg_out, stats_out):
    step = pl.program_id(0)
    hn = _bn(agg_ref[...], stats_ref[...], g_ref[...], bt_ref[...])
    if has_res:
        hn = hn + res_ref[...]
    h = jnp.maximum(hn, 0.0)
    h_out[...] = h
    lin = jax.lax.dot_general(
        h.reshape(G * N, HID).astype(_BF16), w_ref[...].astype(_BF16),
        (((1,), (0,)), ((), ())),
        preferred_element_type=jnp.float32).reshape(G, N, HID)
    a_hat, dinv = _adj_prep(adj_ref[...])
    agg = _aggregate(a_hat, dinv, lin, b_ref[...])
    agg_out[...] = agg
    _acc_stats(step, agg, stats_out)


def _final_kernel(agg_ref, stats_ref, g_ref, bt_ref, res_ref,
                  lng_ref, lnb_ref, out_ref):
    hn = _bn(agg_ref[...], stats_ref[...], g_ref[...], bt_ref[...]) + res_ref[...]
    h = jnp.maximum(hn, 0.0)
    pooled = jnp.mean(h, axis=1)
    mu = jnp.mean(pooled, axis=-1, keepdims=True)
    var = jnp.mean(pooled * pooled, axis=-1, keepdims=True) - mu * mu
    out_ref[...] = (pooled - mu) * jax.lax.rsqrt(var + EPS) * lng_ref[...] \
        + lnb_ref[...]


def _vspec(block, index_map):
    return pl.BlockSpec(block, index_map)


_BLK = lambda i: (i, 0, 0)
_FIX2 = lambda i: (0, 0)

_ADJ_SPEC = pl.BlockSpec((G, N, N), _BLK)
_HG_SPEC = pl.BlockSpec((G, N, HID), _BLK)
_STATS_SPEC = pl.BlockSpec((8, HID), _FIX2)
_ROW_SPEC = pl.BlockSpec((1, HID), _FIX2)

_F32 = jnp.float32


def kernel(adjacency_matrices, in_gamma, in_beta, W0, b0, bn_g0, bn_b0,
           W1, b1, bn_g1, bn_b1, W2, b2, bn_g2, bn_b2, ln_g, ln_b):
    adj = adjacency_matrices
    gamc = in_gamma.reshape(1, 3)
    betc = in_beta.reshape(1, 3)
    row = lambda v: v.reshape(1, HID)

    agg0, st0 = pl.pallas_call(
        _stage1_kernel,
        grid=(STEPS,),
        in_specs=[_ADJ_SPEC,
                  pl.BlockSpec((1, 3), _FIX2),
                  pl.BlockSpec((1, 3), _FIX2),
                  pl.BlockSpec((3, HID), _FIX2),
                  _ROW_SPEC],
        out_specs=[_HG_SPEC, _STATS_SPEC],
        out_shape=[jax.ShapeDtypeStruct((B, N, HID), _F32),
                   jax.ShapeDtypeStruct((8, HID), _F32)],
    )(adj, gamc, betc, W0, row(b0))

    h1, agg1, st1 = pl.pallas_call(
        _mid_kernel_nores,
        grid=(STEPS,),
        in_specs=[_HG_SPEC, _STATS_SPEC, _ROW_SPEC, _ROW_SPEC,
                  _ADJ_SPEC,
                  pl.BlockSpec((HID, HID), _FIX2),
                  _ROW_SPEC],
        out_specs=[_HG_SPEC, _HG_SPEC, _STATS_SPEC],
        out_shape=[jax.ShapeDtypeStruct((B, N, HID), _F32),
                   jax.ShapeDtypeStruct((B, N, HID), _F32),
                   jax.ShapeDtypeStruct((8, HID), _F32)],
    )(agg0, st0, row(bn_g0), row(bn_b0), adj, W1, row(b1))

    h2, agg2, st2 = pl.pallas_call(
        _mid_kernel_res,
        grid=(STEPS,),
        in_specs=[_HG_SPEC, _STATS_SPEC, _ROW_SPEC, _ROW_SPEC,
                  _HG_SPEC,
                  _ADJ_SPEC,
                  pl.BlockSpec((HID, HID), _FIX2),
                  _ROW_SPEC],
        out_specs=[_HG_SPEC, _HG_SPEC, _STATS_SPEC],
        out_shape=[jax.ShapeDtypeStruct((B, N, HID), _F32),
                   jax.ShapeDtypeStruct((B, N, HID), _F32),
                   jax.ShapeDtypeStruct((8, HID), _F32)],
    )(agg1, st1, row(bn_g1), row(bn_b1), h1, adj, W2, row(b2))

    out = pl.pallas_call(
        _final_kernel,
        grid=(STEPS,),
        in_specs=[_HG_SPEC, _STATS_SPEC, _ROW_SPEC, _ROW_SPEC,
                  _HG_SPEC, _ROW_SPEC, _ROW_SPEC],
        out_specs=pl.BlockSpec((G, HID), lambda i: (i, 0)),
        out_shape=jax.ShapeDtypeStruct((B, HID), _F32),
    )(agg2, st2, row(bn_g2), row(bn_b2), h2, row(ln_g), row(ln_b))

    return out


# speedup vs baseline: 1.8296x; 1.1652x over previous
import jax
import jax.numpy as jnp
from jax.experimental import pallas as pl

N_Q, N_X, N_Z = 80, 24, 24
N = N_Q + N_X + N_Z
B = 256
HID = 256
EPS = 1e-5
G = 16
STEPS = B // G
NROWS = B * N

_BF16 = jnp.bfloat16
_F32 = jnp.float32


def _adj_prep(adj_blk):
    a = (adj_blk != 0).astype(_F32)
    row = jax.lax.broadcasted_iota(jnp.int32, (N, N), 0)
    col = jax.lax.broadcasted_iota(jnp.int32, (N, N), 1)
    a_hat = jnp.where((row == col)[None, :, :], 1.0, a)
    deg = jnp.sum(a_hat, axis=1)
    dinv = 1.0 / jnp.sqrt(deg)
    return a_hat, dinv


def _aggregate(a_hat, dinv, lin, bias):
    msg = (dinv[:, :, None] * lin).astype(_BF16)
    agg = jax.lax.dot_general(
        a_hat.astype(_BF16), msg, (((1,), (1,)), ((0,), (0,))),
        preferred_element_type=_F32)
    return agg * dinv[:, :, None] + bias[None]


def _matmul(h, w_ref):
    gg = h.shape[0]
    return jax.lax.dot_general(
        h.reshape(gg * N, HID).astype(_BF16), w_ref[...].astype(_BF16),
        (((1,), (0,)), ((), ())),
        preferred_element_type=_F32).reshape(gg, N, HID)


def _acc_stats(step, agg, stats_out):
    s = jnp.sum(agg, axis=(0, 1))
    q = jnp.sum(agg * agg, axis=(0, 1))
    @pl.when(step == 0)
    def _():
        stats_out[...] = jnp.zeros_like(stats_out)
    stats_out[0:2, :] += jnp.stack([s, q], axis=0)


def _bn(agg, stats, g, bt):
    m = stats[0:1, :] * (1.0 / NROWS)
    q = stats[1:2, :] * (1.0 / NROWS)
    inv = jax.lax.rsqrt((q - m * m) + EPS)
    return (agg - m[None]) * (inv * g)[None] + bt[None]


def _lin0(gam_ref, bet_ref, w0_ref):
    ridx = jax.lax.broadcasted_iota(jnp.int32, (N, 1), 0)
    group = jnp.where(ridx < N_Q, 0, jnp.where(ridx < N_Q + N_X, 1, 2))
    cidx = jax.lax.broadcasted_iota(jnp.int32, (N, 3), 1)
    x = (cidx == group).astype(_F32)
    c3 = jax.lax.broadcasted_iota(jnp.int32, (1, 3), 1)
    p = jnp.where(c3 == 0, N_Q / N,
                  jnp.where(c3 == 1, N_X / N, N_Z / N)).astype(_F32)
    xf = (x - p) / jnp.sqrt(p * (1.0 - p) + EPS) * gam_ref[...] + bet_ref[...]
    return jax.lax.dot_general(
        xf.astype(_BF16), w0_ref[...].astype(_BF16), (((1,), (0,)), ((), ())),
        preferred_element_type=_F32)


def _stage_a(adj_ref, gam_ref, bet_ref, w0_ref, b0_ref, stats_out):
    step = pl.program_id(0)
    lin = _lin0(gam_ref, bet_ref, w0_ref)
    a_hat, dinv = _adj_prep(adj_ref[...])
    agg = _aggregate(a_hat, dinv, jnp.broadcast_to(lin[None], (G, N, HID)),
                     b0_ref[...])
    _acc_stats(step, agg, stats_out)


def _stage_b(adj_ref, gam_ref, bet_ref, w0_ref, b0_ref, st0_ref, g0_ref,
             bt0_ref, w1_ref, b1_ref, h1_out, stats_out):
    step = pl.program_id(0)
    lin = _lin0(gam_ref, bet_ref, w0_ref)
    a_hat, dinv = _adj_prep(adj_ref[...])
    agg0 = _aggregate(a_hat, dinv, jnp.broadcast_to(lin[None], (G, N, HID)),
                      b0_ref[...])
    h1 = jnp.maximum(_bn(agg0, st0_ref[...], g0_ref[...], bt0_ref[...]), 0.0)
    h1_out[...] = h1
    agg1 = _aggregate(a_hat, dinv, _matmul(h1, w1_ref), b1_ref[...])
    _acc_stats(step, agg1, stats_out)


def _stage_c(adj_ref, h1_ref, w1_ref, b1_ref, st1_ref, g1_ref, bt1_ref,
             w2_ref, b2_ref, h2_out, stats_out):
    step = pl.program_id(0)
    a_hat, dinv = _adj_prep(adj_ref[...])
    h1 = h1_ref[...]
    agg1 = _aggregate(a_hat, dinv, _matmul(h1, w1_ref), b1_ref[...])
    h2 = jnp.maximum(
        _bn(agg1, st1_ref[...], g1_ref[...], bt1_ref[...]) + h1, 0.0)
    h2_out[...] = h2
    agg2 = _aggregate(a_hat, dinv, _matmul(h2, w2_ref), b2_ref[...])
    _acc_stats(step, agg2, stats_out)


def _stage_d(adj_ref, h2_ref, w2_ref, b2_ref, st2_ref, g2_ref, bt2_ref,
             lng_ref, lnb_ref, out_ref):
    a_hat, dinv = _adj_prep(adj_ref[...])
    h2 = h2_ref[...]
    agg2 = _aggregate(a_hat, dinv, _matmul(h2, w2_ref), b2_ref[...])
    h3 = jnp.maximum(
        _bn(agg2, st2_ref[...], g2_ref[...], bt2_ref[...]) + h2, 0.0)
    pooled = jnp.mean(h3, axis=1)
    mu = jnp.mean(pooled, axis=-1, keepdims=True)
    var = jnp.mean(pooled * pooled, axis=-1, keepdims=True) - mu * mu
    out_ref[...] = (pooled - mu) * jax.lax.rsqrt(var + EPS) * lng_ref[...] \
        + lnb_ref[...]


_BLK = lambda i: (i, 0, 0)
_FIX2 = lambda i: (0, 0)

_ADJ_SPEC = pl.BlockSpec((G, N, N), _BLK)
_HG_SPEC = pl.BlockSpec((G, N, HID), _BLK)
_STATS_SPEC = pl.BlockSpec((8, HID), _FIX2)
_ROW_SPEC = pl.BlockSpec((1, HID), _FIX2)
_W_SPEC = pl.BlockSpec((HID, HID), _FIX2)

_STATS_OUT = jax.ShapeDtypeStruct((8, HID), _F32)
_H_OUT = jax.ShapeDtypeStruct((B, N, HID), _F32)


def kernel(adjacency_matrices, in_gamma, in_beta, W0, b0, bn_g0, bn_b0,
           W1, b1, bn_g1, bn_b1, W2, b2, bn_g2, bn_b2, ln_g, ln_b):
    adj = adjacency_matrices.astype(jnp.int8)
    gamc = in_gamma.reshape(1, 3)
    betc = in_beta.reshape(1, 3)
    row = lambda v: v.reshape(1, HID)
    w0spec = pl.BlockSpec((3, HID), _FIX2)
    g3spec = pl.BlockSpec((1, 3), _FIX2)

    st0 = pl.pallas_call(
        _stage_a,
        grid=(STEPS,),
        in_specs=[_ADJ_SPEC, g3spec, g3spec, w0spec, _ROW_SPEC],
        out_specs=_STATS_SPEC,
        out_shape=_STATS_OUT,
    )(adj, gamc, betc, W0, row(b0))

    h1, st1 = pl.pallas_call(
        _stage_b,
        grid=(STEPS,),
        in_specs=[_ADJ_SPEC, g3spec, g3spec, w0spec, _ROW_SPEC,
                  _STATS_SPEC, _ROW_SPEC, _ROW_SPEC, _W_SPEC, _ROW_SPEC],
        out_specs=[_HG_SPEC, _STATS_SPEC],
        out_shape=[_H_OUT, _STATS_OUT],
    )(adj, gamc, betc, W0, row(b0), st0, row(bn_g0), row(bn_b0), W1, row(b1))

    h2, st2 = pl.pallas_call(
        _stage_c,
        grid=(STEPS,),
        in_specs=[_ADJ_SPEC, _HG_SPEC, _W_SPEC, _ROW_SPEC,
                  _STATS_SPEC, _ROW_SPEC, _ROW_SPEC, _W_SPEC, _ROW_SPEC],
        out_specs=[_HG_SPEC, _STATS_SPEC],
        out_shape=[_H_OUT, _STATS_OUT],
    )(adj, h1, W1, row(b1), st1, row(bn_g1), row(bn_b1), W2, row(b2))

    out = pl.pallas_call(
        _stage_d,
        grid=(STEPS,),
        in_specs=[_ADJ_SPEC, _HG_SPEC, _W_SPEC, _ROW_SPEC,
                  _STATS_SPEC, _ROW_SPEC, _ROW_SPEC, _ROW_SPEC, _ROW_SPEC],
        out_specs=pl.BlockSpec((G, HID), lambda i: (i, 0)),
        out_shape=jax.ShapeDtypeStruct((B, HID), _F32),
    )(adj, h2, W2, row(b2), st2, row(bn_g2), row(bn_b2), row(ln_g), row(ln_b))

    return out


# speedup vs baseline: 1.8963x; 1.0365x over previous
import jax
import jax.numpy as jnp
from jax.experimental import pallas as pl

N_Q, N_X, N_Z = 80, 24, 24
N = N_Q + N_X + N_Z
B = 256
HID = 256
EPS = 1e-5
G = 16
STEPS = B // G
NROWS = B * N

_BF16 = jnp.bfloat16
_F32 = jnp.float32


def _adj_prep(adj_blk):
    a = (adj_blk != 0).astype(_F32)
    row = jax.lax.broadcasted_iota(jnp.int32, (N, N), 0)
    col = jax.lax.broadcasted_iota(jnp.int32, (N, N), 1)
    a_hat = jnp.where((row == col)[None, :, :], 1.0, a)
    deg = jnp.sum(a_hat, axis=1)
    dinv = 1.0 / jnp.sqrt(deg)
    return a_hat, dinv


def _aggregate(ah16, dinv, lin, bias):
    msg = (dinv[:, :, None] * lin).astype(_BF16)
    agg = jax.lax.dot_general(
        ah16, msg, (((1,), (1,)), ((0,), (0,))),
        preferred_element_type=_F32)
    return agg * dinv[:, :, None] + bias[None]


def _matmul(h, w_ref):
    gg = h.shape[0]
    return jax.lax.dot_general(
        h.reshape(gg * N, HID).astype(_BF16), w_ref[...].astype(_BF16),
        (((1,), (0,)), ((), ())),
        preferred_element_type=_F32).reshape(gg, N, HID)


def _acc_stats(step, agg, stats_out):
    s = jnp.sum(agg, axis=(0, 1))
    q = jnp.sum(agg * agg, axis=(0, 1))
    @pl.when(step == 0)
    def _():
        stats_out[...] = jnp.zeros_like(stats_out)
    stats_out[0:2, :] += jnp.stack([s, q], axis=0)


def _bn(agg, stats, g, bt):
    m = stats[0:1, :] * (1.0 / NROWS)
    q = stats[1:2, :] * (1.0 / NROWS)
    inv = jax.lax.rsqrt((q - m * m) + EPS)
    return (agg - m[None]) * (inv * g)[None] + bt[None]


def _lin0(gam_ref, bet_ref, w0_ref):
    ridx = jax.lax.broadcasted_iota(jnp.int32, (N, 1), 0)
    group = jnp.where(ridx < N_Q, 0, jnp.where(ridx < N_Q + N_X, 1, 2))
    cidx = jax.lax.broadcasted_iota(jnp.int32, (N, 3), 1)
    x = (cidx == group).astype(_F32)
    c3 = jax.lax.broadcasted_iota(jnp.int32, (1, 3), 1)
    p = jnp.where(c3 == 0, N_Q / N,
                  jnp.where(c3 == 1, N_X / N, N_Z / N)).astype(_F32)
    xf = (x - p) / jnp.sqrt(p * (1.0 - p) + EPS) * gam_ref[...] + bet_ref[...]
    return jax.lax.dot_general(
        xf.astype(_BF16), w0_ref[...].astype(_BF16), (((1,), (0,)), ((), ())),
        preferred_element_type=_F32)


def _stage_a(adj_ref, gam_ref, bet_ref, w0_ref, b0_ref,
             ah_out, dinv_out, stats_out):
    step = pl.program_id(0)
    lin = _lin0(gam_ref, bet_ref, w0_ref)
    a_hat, dinv = _adj_prep(adj_ref[...])
    ah16 = a_hat.astype(_BF16)
    ah_out[...] = ah16
    dinv_out[...] = dinv
    agg = _aggregate(ah16, dinv, jnp.broadcast_to(lin[None], (G, N, HID)),
                     b0_ref[...])
    _acc_stats(step, agg, stats_out)


def _stage_b(ah_ref, dinv_ref, gam_ref, bet_ref, w0_ref, b0_ref, st0_ref,
             g0_ref, bt0_ref, w1_ref, b1_ref, h1_out, stats_out):
    step = pl.program_id(0)
    lin = _lin0(gam_ref, bet_ref, w0_ref)
    ah16, dinv = ah_ref[...], dinv_ref[...]
    agg0 = _aggregate(ah16, dinv, jnp.broadcast_to(lin[None], (G, N, HID)),
                      b0_ref[...])
    h1 = jnp.maximum(_bn(agg0, st0_ref[...], g0_ref[...], bt0_ref[...]), 0.0)
    h1_out[...] = h1
    agg1 = _aggregate(ah16, dinv, _matmul(h1, w1_ref), b1_ref[...])
    _acc_stats(step, agg1, stats_out)


def _stage_c(ah_ref, dinv_ref, h1_ref, w1_ref, b1_ref, st1_ref, g1_ref,
             bt1_ref, w2_ref, b2_ref, h2_out, stats_out):
    step = pl.program_id(0)
    ah16, dinv = ah_ref[...], dinv_ref[...]
    h1 = h1_ref[...]
    agg1 = _aggregate(ah16, dinv, _matmul(h1, w1_ref), b1_ref[...])
    h2 = jnp.maximum(
        _bn(agg1, st1_ref[...], g1_ref[...], bt1_ref[...]) + h1, 0.0)
    h2_out[...] = h2
    agg2 = _aggregate(ah16, dinv, _matmul(h2, w2_ref), b2_ref[...])
    _acc_stats(step, agg2, stats_out)


def _stage_d(ah_ref, dinv_ref, h2_ref, w2_ref, b2_ref, st2_ref, g2_ref,
             bt2_ref, lng_ref, lnb_ref, out_ref):
    ah16, dinv = ah_ref[...], dinv_ref[...]
    h2 = h2_ref[...]
    agg2 = _aggregate(ah16, dinv, _matmul(h2, w2_ref), b2_ref[...])
    h3 = jnp.maximum(
        _bn(agg2, st2_ref[...], g2_ref[...], bt2_ref[...]) + h2, 0.0)
    pooled = jnp.mean(h3, axis=1)
    mu = jnp.mean(pooled, axis=-1, keepdims=True)
    var = jnp.mean(pooled * pooled, axis=-1, keepdims=True) - mu * mu
    out_ref[...] = (pooled - mu) * jax.lax.rsqrt(var + EPS) * lng_ref[...] \
        + lnb_ref[...]


_BLK = lambda i: (i, 0, 0)
_FIX2 = lambda i: (0, 0)

_ADJ_SPEC = pl.BlockSpec((G, N, N), _BLK)
_HG_SPEC = pl.BlockSpec((G, N, HID), _BLK)
_STATS_SPEC = pl.BlockSpec((8, HID), _FIX2)
_ROW_SPEC = pl.BlockSpec((1, HID), _FIX2)
_W_SPEC = pl.BlockSpec((HID, HID), _FIX2)

_STATS_OUT = jax.ShapeDtypeStruct((8, HID), _F32)
_H_OUT = jax.ShapeDtypeStruct((B, N, HID), _F32)


def kernel(adjacency_matrices, in_gamma, in_beta, W0, b0, bn_g0, bn_b0,
           W1, b1, bn_g1, bn_b1, W2, b2, bn_g2, bn_b2, ln_g, ln_b):
    adj = adjacency_matrices.astype(jnp.int8)
    gamc = in_gamma.reshape(1, 3)
    betc = in_beta.reshape(1, 3)
    row = lambda v: v.reshape(1, HID)
    w0spec = pl.BlockSpec((3, HID), _FIX2)
    g3spec = pl.BlockSpec((1, 3), _FIX2)

    _DINV_SPEC = pl.BlockSpec((G, N), lambda i: (i, 0))

    ah, dinv, st0 = pl.pallas_call(
        _stage_a,
        grid=(STEPS,),
        in_specs=[_ADJ_SPEC, g3spec, g3spec, w0spec, _ROW_SPEC],
        out_specs=[_ADJ_SPEC, _DINV_SPEC, _STATS_SPEC],
        out_shape=[jax.ShapeDtypeStruct((B, N, N), _BF16),
                   jax.ShapeDtypeStruct((B, N), _F32),
                   _STATS_OUT],
    )(adj, gamc, betc, W0, row(b0))

    h1, st1 = pl.pallas_call(
        _stage_b,
        grid=(STEPS,),
        in_specs=[_ADJ_SPEC, _DINV_SPEC, g3spec, g3spec, w0spec, _ROW_SPEC,
                  _STATS_SPEC, _ROW_SPEC, _ROW_SPEC, _W_SPEC, _ROW_SPEC],
        out_specs=[_HG_SPEC, _STATS_SPEC],
        out_shape=[_H_OUT, _STATS_OUT],
    )(ah, dinv, gamc, betc, W0, row(b0), st0, row(bn_g0), row(bn_b0),
      W1, row(b1))

    h2, st2 = pl.pallas_call(
        _stage_c,
        grid=(STEPS,),
        in_specs=[_ADJ_SPEC, _DINV_SPEC, _HG_SPEC, _W_SPEC, _ROW_SPEC,
                  _STATS_SPEC, _ROW_SPEC, _ROW_SPEC, _W_SPEC, _ROW_SPEC],
        out_specs=[_HG_SPEC, _STATS_SPEC],
        out_shape=[_H_OUT, _STATS_OUT],
    )(ah, dinv, h1, W1, row(b1), st1, row(bn_g1), row(bn_b1), W2, row(b2))

    out = pl.pallas_call(
        _stage_d,
        grid=(STEPS,),
        in_specs=[_ADJ_SPEC, _DINV_SPEC, _HG_SPEC, _W_SPEC, _ROW_SPEC,
                  _STATS_SPEC, _ROW_SPEC, _ROW_SPEC, _ROW_SPEC, _ROW_SPEC],
        out_specs=pl.BlockSpec((G, HID), lambda i: (i, 0)),
        out_shape=jax.ShapeDtypeStruct((B, HID), _F32),
    )(ah, dinv, h2, W2, row(b2), st2, row(bn_g2), row(bn_b2),
      row(ln_g), row(ln_b))

    return out


# speedup vs baseline: 2.0920x; 1.1032x over previous
import jax
import jax.numpy as jnp
from jax.experimental import pallas as pl

N_Q, N_X, N_Z = 80, 24, 24
N = N_Q + N_X + N_Z
B = 256
HID = 256
EPS = 1e-5
G = 32
STEPS = B // G
NROWS = B * N

_BF16 = jnp.bfloat16
_F32 = jnp.float32


def _adj_prep(adj_blk):
    a = (adj_blk != 0).astype(_F32)
    row = jax.lax.broadcasted_iota(jnp.int32, (N, N), 0)
    col = jax.lax.broadcasted_iota(jnp.int32, (N, N), 1)
    a_hat = jnp.where((row == col)[None, :, :], 1.0, a)
    deg = jnp.sum(a_hat, axis=1)
    dinv = 1.0 / jnp.sqrt(deg)
    return a_hat, dinv


def _aggregate(ah16, dinv, lin, bias):
    msg = (dinv[:, :, None] * lin).astype(_BF16)
    agg = jax.lax.dot_general(
        ah16, msg, (((1,), (1,)), ((0,), (0,))),
        preferred_element_type=_F32)
    return agg * dinv[:, :, None] + bias[None]


def _matmul(h, w_ref):
    gg = h.shape[0]
    return jax.lax.dot_general(
        h.reshape(gg * N, HID).astype(_BF16), w_ref[...].astype(_BF16),
        (((1,), (0,)), ((), ())),
        preferred_element_type=_F32).reshape(gg, N, HID)


def _acc_stats(step, agg, stats_out):
    s = jnp.sum(agg, axis=(0, 1))
    q = jnp.sum(agg * agg, axis=(0, 1))
    @pl.when(step == 0)
    def _():
        stats_out[...] = jnp.zeros_like(stats_out)
    stats_out[0:2, :] += jnp.stack([s, q], axis=0)


def _bn(agg, stats, g, bt):
    m = stats[0:1, :] * (1.0 / NROWS)
    q = stats[1:2, :] * (1.0 / NROWS)
    inv = jax.lax.rsqrt((q - m * m) + EPS)
    return (agg - m[None]) * (inv * g)[None] + bt[None]


def _lin0(gam_ref, bet_ref, w0_ref):
    ridx = jax.lax.broadcasted_iota(jnp.int32, (N, 1), 0)
    group = jnp.where(ridx < N_Q, 0, jnp.where(ridx < N_Q + N_X, 1, 2))
    cidx = jax.lax.broadcasted_iota(jnp.int32, (N, 3), 1)
    x = (cidx == group).astype(_F32)
    c3 = jax.lax.broadcasted_iota(jnp.int32, (1, 3), 1)
    p = jnp.where(c3 == 0, N_Q / N,
                  jnp.where(c3 == 1, N_X / N, N_Z / N)).astype(_F32)
    xf = (x - p) / jnp.sqrt(p * (1.0 - p) + EPS) * gam_ref[...] + bet_ref[...]
    return jax.lax.dot_general(
        xf.astype(_BF16), w0_ref[...].astype(_BF16), (((1,), (0,)), ((), ())),
        preferred_element_type=_F32)


def _stage_a(adj_ref, gam_ref, bet_ref, w0_ref, b0_ref,
             ah_out, dinv_out, stats_out):
    step = pl.program_id(0)
    lin = _lin0(gam_ref, bet_ref, w0_ref)
    a_hat, dinv = _adj_prep(adj_ref[...])
    ah16 = a_hat.astype(_BF16)
    ah_out[...] = ah16
    dinv_out[...] = dinv
    agg = _aggregate(ah16, dinv, jnp.broadcast_to(lin[None], (G, N, HID)),
                     b0_ref[...])
    _acc_stats(step, agg, stats_out)


def _stage_b(ah_ref, dinv_ref, gam_ref, bet_ref, w0_ref, b0_ref, st0_ref,
             g0_ref, bt0_ref, w1_ref, b1_ref, h1_out, stats_out):
    step = pl.program_id(0)
    lin = _lin0(gam_ref, bet_ref, w0_ref)
    ah16, dinv = ah_ref[...], dinv_ref[...]
    agg0 = _aggregate(ah16, dinv, jnp.broadcast_to(lin[None], (G, N, HID)),
                      b0_ref[...])
    h1 = jnp.maximum(_bn(agg0, st0_ref[...], g0_ref[...], bt0_ref[...]), 0.0)
    h1_out[...] = h1
    agg1 = _aggregate(ah16, dinv, _matmul(h1, w1_ref), b1_ref[...])
    _acc_stats(step, agg1, stats_out)


def _stage_c(ah_ref, dinv_ref, h1_ref, w1_ref, b1_ref, st1_ref, g1_ref,
             bt1_ref, w2_ref, b2_ref, h2_out, stats_out):
    step = pl.program_id(0)
    ah16, dinv = ah_ref[...], dinv_ref[...]
    h1 = h1_ref[...]
    agg1 = _aggregate(ah16, dinv, _matmul(h1, w1_ref), b1_ref[...])
    h2 = jnp.maximum(
        _bn(agg1, st1_ref[...], g1_ref[...], bt1_ref[...]) + h1, 0.0)
    h2_out[...] = h2
    agg2 = _aggregate(ah16, dinv, _matmul(h2, w2_ref), b2_ref[...])
    _acc_stats(step, agg2, stats_out)


def _stage_d(ah_ref, dinv_ref, h2_ref, w2_ref, b2_ref, st2_ref, g2_ref,
             bt2_ref, lng_ref, lnb_ref, out_ref):
    ah16, dinv = ah_ref[...], dinv_ref[...]
    h2 = h2_ref[...]
    agg2 = _aggregate(ah16, dinv, _matmul(h2, w2_ref), b2_ref[...])
    h3 = jnp.maximum(
        _bn(agg2, st2_ref[...], g2_ref[...], bt2_ref[...]) + h2, 0.0)
    pooled = jnp.mean(h3, axis=1)
    mu = jnp.mean(pooled, axis=-1, keepdims=True)
    var = jnp.mean(pooled * pooled, axis=-1, keepdims=True) - mu * mu
    out_ref[...] = (pooled - mu) * jax.lax.rsqrt(var + EPS) * lng_ref[...] \
        + lnb_ref[...]


_BLK = lambda i: (i, 0, 0)
_FIX2 = lambda i: (0, 0)

_ADJ_SPEC = pl.BlockSpec((G, N, N), _BLK)
_HG_SPEC = pl.BlockSpec((G, N, HID), _BLK)
_STATS_SPEC = pl.BlockSpec((8, HID), _FIX2)
_ROW_SPEC = pl.BlockSpec((1, HID), _FIX2)
_W_SPEC = pl.BlockSpec((HID, HID), _FIX2)

_STATS_OUT = jax.ShapeDtypeStruct((8, HID), _F32)
_H_OUT = jax.ShapeDtypeStruct((B, N, HID), _F32)


def kernel(adjacency_matrices, in_gamma, in_beta, W0, b0, bn_g0, bn_b0,
           W1, b1, bn_g1, bn_b1, W2, b2, bn_g2, bn_b2, ln_g, ln_b):
    adj = adjacency_matrices.astype(jnp.int8)
    gamc = in_gamma.reshape(1, 3)
    betc = in_beta.reshape(1, 3)
    row = lambda v: v.reshape(1, HID)
    w0spec = pl.BlockSpec((3, HID), _FIX2)
    g3spec = pl.BlockSpec((1, 3), _FIX2)

    _DINV_SPEC = pl.BlockSpec((G, N), lambda i: (i, 0))

    ah, dinv, st0 = pl.pallas_call(
        _stage_a,
        grid=(STEPS,),
        in_specs=[_ADJ_SPEC, g3spec, g3spec, w0spec, _ROW_SPEC],
        out_specs=[_ADJ_SPEC, _DINV_SPEC, _STATS_SPEC],
        out_shape=[jax.ShapeDtypeStruct((B, N, N), _BF16),
                   jax.ShapeDtypeStruct((B, N), _F32),
                   _STATS_OUT],
    )(adj, gamc, betc, W0, row(b0))

    h1, st1 = pl.pallas_call(
        _stage_b,
        grid=(STEPS,),
        in_specs=[_ADJ_SPEC, _DINV_SPEC, g3spec, g3spec, w0spec, _ROW_SPEC,
                  _STATS_SPEC, _ROW_SPEC, _ROW_SPEC, _W_SPEC, _ROW_SPEC],
        out_specs=[_HG_SPEC, _STATS_SPEC],
        out_shape=[_H_OUT, _STATS_OUT],
    )(ah, dinv, gamc, betc, W0, row(b0), st0, row(bn_g0), row(bn_b0),
      W1, row(b1))

    h2, st2 = pl.pallas_call(
        _stage_c,
        grid=(STEPS,),
        in_specs=[_ADJ_SPEC, _DINV_SPEC, _HG_SPEC, _W_SPEC, _ROW_SPEC,
                  _STATS_SPEC, _ROW_SPEC, _ROW_SPEC, _W_SPEC, _ROW_SPEC],
        out_specs=[_HG_SPEC, _STATS_SPEC],
        out_shape=[_H_OUT, _STATS_OUT],
    )(ah, dinv, h1, W1, row(b1), st1, row(bn_g1), row(bn_b1), W2, row(b2))

    out = pl.pallas_call(
        _stage_d,
        grid=(STEPS,),
        in_specs=[_ADJ_SPEC, _DINV_SPEC, _HG_SPEC, _W_SPEC, _ROW_SPEC,
                  _STATS_SPEC, _ROW_SPEC, _ROW_SPEC, _ROW_SPEC, _ROW_SPEC],
        out_specs=pl.BlockSpec((G, HID), lambda i: (i, 0)),
        out_shape=jax.ShapeDtypeStruct((B, HID), _F32),
    )(ah, dinv, h2, W2, row(b2), st2, row(bn_g2), row(bn_b2),
      row(ln_g), row(ln_b))

    return out


# speedup vs baseline: 2.1535x; 1.0294x over previous
import jax
import jax.numpy as jnp
from jax.experimental import pallas as pl

N_Q, N_X, N_Z = 80, 24, 24
N = N_Q + N_X + N_Z
B = 256
HID = 256
EPS = 1e-5
G = 32
STEPS = B // G
NROWS = B * N

_BF16 = jnp.bfloat16
_F32 = jnp.float32


def _adj_prep(adj_blk):
    a = (adj_blk != 0).astype(_F32)
    row = jax.lax.broadcasted_iota(jnp.int32, (N, N), 0)
    col = jax.lax.broadcasted_iota(jnp.int32, (N, N), 1)
    a_hat = jnp.where((row == col)[None, :, :], 1.0, a)
    deg = jnp.sum(a_hat, axis=1)
    dinv = 1.0 / jnp.sqrt(deg)
    return a_hat, dinv


def _aggregate(ah16, dinv, lin, bias):
    msg = (dinv[:, :, None] * lin).astype(_BF16)
    agg = jax.lax.dot_general(
        ah16, msg, (((1,), (1,)), ((0,), (0,))),
        preferred_element_type=_F32)
    return agg * dinv[:, :, None] + bias[None]


def _matmul(h, w_ref):
    gg = h.shape[0]
    return jax.lax.dot_general(
        h.reshape(gg * N, HID).astype(_BF16), w_ref[...].astype(_BF16),
        (((1,), (0,)), ((), ())),
        preferred_element_type=_F32).reshape(gg, N, HID)


def _acc_stats(step, agg, stats_out):
    s = jnp.sum(agg, axis=(0, 1))
    q = jnp.sum(agg * agg, axis=(0, 1))
    @pl.when(step == 0)
    def _():
        stats_out[...] = jnp.zeros_like(stats_out)
    stats_out[0:2, :] += jnp.stack([s, q], axis=0)


def _bn(agg, stats, g, bt):
    m = stats[0:1, :] * (1.0 / NROWS)
    q = stats[1:2, :] * (1.0 / NROWS)
    inv = jax.lax.rsqrt((q - m * m) + EPS)
    return (agg - m[None]) * (inv * g)[None] + bt[None]


def _lin0(gam_ref, bet_ref, w0_ref):
    ridx = jax.lax.broadcasted_iota(jnp.int32, (N, 1), 0)
    group = jnp.where(ridx < N_Q, 0, jnp.where(ridx < N_Q + N_X, 1, 2))
    cidx = jax.lax.broadcasted_iota(jnp.int32, (N, 3), 1)
    x = (cidx == group).astype(_F32)
    c3 = jax.lax.broadcasted_iota(jnp.int32, (1, 3), 1)
    p = jnp.where(c3 == 0, N_Q / N,
                  jnp.where(c3 == 1, N_X / N, N_Z / N)).astype(_F32)
    xf = (x - p) / jnp.sqrt(p * (1.0 - p) + EPS) * gam_ref[...] + bet_ref[...]
    return jax.lax.dot_general(
        xf.astype(_BF16), w0_ref[...].astype(_BF16), (((1,), (0,)), ((), ())),
        preferred_element_type=_F32)


def _stage_a(adj_ref, gam_ref, bet_ref, w0_ref, b0_ref,
             ah_out, dinv_out, stats_out):
    step = pl.program_id(0)
    lin = _lin0(gam_ref, bet_ref, w0_ref)
    a_hat, dinv = _adj_prep(adj_ref[...])
    ah16 = a_hat.astype(_BF16)
    ah_out[...] = ah16
    dinv_out[...] = dinv
    agg = _aggregate(ah16, dinv, jnp.broadcast_to(lin[None], (G, N, HID)),
                     b0_ref[...])
    _acc_stats(step, agg, stats_out)


def _stage_b(ah_ref, dinv_ref, gam_ref, bet_ref, w0_ref, b0_ref, st0_ref,
             g0_ref, bt0_ref, w1_ref, b1_ref, h1_out, stats_out):
    step = pl.program_id(0)
    lin = _lin0(gam_ref, bet_ref, w0_ref)
    ah16, dinv = ah_ref[...], dinv_ref[...]
    agg0 = _aggregate(ah16, dinv, jnp.broadcast_to(lin[None], (G, N, HID)),
                      b0_ref[...])
    h1 = jnp.maximum(_bn(agg0, st0_ref[...], g0_ref[...], bt0_ref[...]), 0.0)
    h1_out[...] = h1.astype(_BF16)
    agg1 = _aggregate(ah16, dinv, _matmul(h1, w1_ref), b1_ref[...])
    _acc_stats(step, agg1, stats_out)


def _stage_c(ah_ref, dinv_ref, h1_ref, w1_ref, b1_ref, st1_ref, g1_ref,
             bt1_ref, w2_ref, b2_ref, h2_out, stats_out):
    step = pl.program_id(0)
    ah16, dinv = ah_ref[...], dinv_ref[...]
    h1 = h1_ref[...].astype(_F32)
    agg1 = _aggregate(ah16, dinv, _matmul(h1, w1_ref), b1_ref[...])
    h2 = jnp.maximum(
        _bn(agg1, st1_ref[...], g1_ref[...], bt1_ref[...]) + h1, 0.0)
    h2_out[...] = h2.astype(_BF16)
    agg2 = _aggregate(ah16, dinv, _matmul(h2, w2_ref), b2_ref[...])
    _acc_stats(step, agg2, stats_out)


def _stage_d(ah_ref, dinv_ref, h2_ref, w2_ref, b2_ref, st2_ref, g2_ref,
             bt2_ref, lng_ref, lnb_ref, out_ref):
    ah16, dinv = ah_ref[...], dinv_ref[...]
    h2 = h2_ref[...].astype(_F32)
    agg2 = _aggregate(ah16, dinv, _matmul(h2, w2_ref), b2_ref[...])
    h3 = jnp.maximum(
        _bn(agg2, st2_ref[...], g2_ref[...], bt2_ref[...]) + h2, 0.0)
    pooled = jnp.mean(h3, axis=1)
    mu = jnp.mean(pooled, axis=-1, keepdims=True)
    var = jnp.mean(pooled * pooled, axis=-1, keepdims=True) - mu * mu
    out_ref[...] = (pooled - mu) * jax.lax.rsqrt(var + EPS) * lng_ref[...] \
        + lnb_ref[...]


_BLK = lambda i: (i, 0, 0)
_FIX2 = lambda i: (0, 0)

_ADJ_SPEC = pl.BlockSpec((G, N, N), _BLK)
_HG_SPEC = pl.BlockSpec((G, N, HID), _BLK)
_STATS_SPEC = pl.BlockSpec((8, HID), _FIX2)
_ROW_SPEC = pl.BlockSpec((1, HID), _FIX2)
_W_SPEC = pl.BlockSpec((HID, HID), _FIX2)

_STATS_OUT = jax.ShapeDtypeStruct((8, HID), _F32)
_H_OUT = jax.ShapeDtypeStruct((B, N, HID), _BF16)


def kernel(adjacency_matrices, in_gamma, in_beta, W0, b0, bn_g0, bn_b0,
           W1, b1, bn_g1, bn_b1, W2, b2, bn_g2, bn_b2, ln_g, ln_b):
    adj = adjacency_matrices.astype(jnp.int8)
    gamc = in_gamma.reshape(1, 3)
    betc = in_beta.reshape(1, 3)
    row = lambda v: v.reshape(1, HID)
    w0spec = pl.BlockSpec((3, HID), _FIX2)
    g3spec = pl.BlockSpec((1, 3), _FIX2)

    _DINV_SPEC = pl.BlockSpec((G, N), lambda i: (i, 0))

    ah, dinv, st0 = pl.pallas_call(
        _stage_a,
        grid=(STEPS,),
        in_specs=[_ADJ_SPEC, g3spec, g3spec, w0spec, _ROW_SPEC],
        out_specs=[_ADJ_SPEC, _DINV_SPEC, _STATS_SPEC],
        out_shape=[jax.ShapeDtypeStruct((B, N, N), _BF16),
                   jax.ShapeDtypeStruct((B, N), _F32),
                   _STATS_OUT],
    )(adj, gamc, betc, W0, row(b0))

    h1, st1 = pl.pallas_call(
        _stage_b,
        grid=(STEPS,),
        in_specs=[_ADJ_SPEC, _DINV_SPEC, g3spec, g3spec, w0spec, _ROW_SPEC,
                  _STATS_SPEC, _ROW_SPEC, _ROW_SPEC, _W_SPEC, _ROW_SPEC],
        out_specs=[_HG_SPEC, _STATS_SPEC],
        out_shape=[_H_OUT, _STATS_OUT],
    )(ah, dinv, gamc, betc, W0, row(b0), st0, row(bn_g0), row(bn_b0),
      W1, row(b1))

    h2, st2 = pl.pallas_call(
        _stage_c,
        grid=(STEPS,),
        in_specs=[_ADJ_SPEC, _DINV_SPEC, _HG_SPEC, _W_SPEC, _ROW_SPEC,
                  _STATS_SPEC, _ROW_SPEC, _ROW_SPEC, _W_SPEC, _ROW_SPEC],
        out_specs=[_HG_SPEC, _STATS_SPEC],
        out_shape=[_H_OUT, _STATS_OUT],
    )(ah, dinv, h1, W1, row(b1), st1, row(bn_g1), row(bn_b1), W2, row(b2))

    out = pl.pallas_call(
        _stage_d,
        grid=(STEPS,),
        in_specs=[_ADJ_SPEC, _DINV_SPEC, _HG_SPEC, _W_SPEC, _ROW_SPEC,
                  _STATS_SPEC, _ROW_SPEC, _ROW_SPEC, _ROW_SPEC, _ROW_SPEC],
        out_specs=pl.BlockSpec((G, HID), lambda i: (i, 0)),
        out_shape=jax.ShapeDtypeStruct((B, HID), _F32),
    )(ah, dinv, h2, W2, row(b2), st2, row(bn_g2), row(bn_b2),
      row(ln_g), row(ln_b))

    return out


# speedup vs baseline: 2.5226x; 1.1714x over previous
import jax
import jax.numpy as jnp
from jax.experimental import pallas as pl
from jax.experimental.pallas import tpu as pltpu

N_Q, N_X, N_Z = 80, 24, 24
N = N_Q + N_X + N_Z
B = 256
HID = 256
EPS = 1e-5
G = 32
STEPS = B // G
NROWS = B * N

_BF16 = jnp.bfloat16
_F32 = jnp.float32


def _adj_prep(adj_blk):
    a = (adj_blk != 0).astype(_F32)
    row = jax.lax.broadcasted_iota(jnp.int32, (N, N), 0)
    col = jax.lax.broadcasted_iota(jnp.int32, (N, N), 1)
    a_hat = jnp.where((row == col)[None, :, :], 1.0, a)
    deg = jnp.sum(a_hat, axis=1)
    dinv = 1.0 / jnp.sqrt(deg)
    return a_hat, dinv


def _agg_raw(ah16, dinv, lin):
    msg = (dinv[:, :, None] * lin).astype(_BF16)
    mm = jax.lax.dot_general(
        ah16, msg, (((1,), (1,)), ((0,), (0,))),
        preferred_element_type=_F32)
    return mm * dinv[:, :, None]


def _matmul(h, w_ref):
    gg = h.shape[0]
    return jax.lax.dot_general(
        h.reshape(gg * N, HID).astype(_BF16), w_ref[...].astype(_BF16),
        (((1,), (0,)), ((), ())),
        preferred_element_type=_F32).reshape(gg, N, HID)


def _acc_stats(i, raw, bias, st_scr, r):
    agg = raw + bias[None]
    s = jnp.sum(agg, axis=(0, 1))
    q = jnp.sum(agg * agg, axis=(0, 1))
    @pl.when(i == 0)
    def _():
        st_scr[r:r + 2, :] = jnp.zeros((2, HID), _F32)
    st_scr[r:r + 2, :] += jnp.stack([s, q], axis=0)


def _bn_coeffs(st_scr, r, bias, g, bt):
    m = st_scr[r:r + 1, :] * (1.0 / NROWS)
    q = st_scr[r + 1:r + 2, :] * (1.0 / NROWS)
    sv = jax.lax.rsqrt((q - m * m) + EPS) * g
    cv = (bias - m) * sv + bt
    return sv, cv


def _lin0(gam_ref, bet_ref, w0_ref):
    ridx = jax.lax.broadcasted_iota(jnp.int32, (N, 1), 0)
    group = jnp.where(ridx < N_Q, 0, jnp.where(ridx < N_Q + N_X, 1, 2))
    cidx = jax.lax.broadcasted_iota(jnp.int32, (N, 3), 1)
    x = (cidx == group).astype(_F32)
    c3 = jax.lax.broadcasted_iota(jnp.int32, (1, 3), 1)
    p = jnp.where(c3 == 0, N_Q / N,
                  jnp.where(c3 == 1, N_X / N, N_Z / N)).astype(_F32)
    xf = (x - p) / jnp.sqrt(p * (1.0 - p) + EPS) * gam_ref[...] + bet_ref[...]
    return jax.lax.dot_general(
        xf.astype(_BF16), w0_ref[...].astype(_BF16), (((1,), (0,)), ((), ())),
        preferred_element_type=_F32)


def _fused(adj_ref, gam_ref, bet_ref, w0_ref, b0_ref, g0_ref, bt0_ref,
           w1_ref, b1_ref, g1_ref, bt1_ref, w2_ref, b2_ref, g2_ref, bt2_ref,
           lng_ref, lnb_ref, out_ref,
           ah_scr, dinv_scr, h1_scr, h2_scr, st_scr):
    s = pl.program_id(0)
    i = pl.program_id(1)
    sl = pl.ds(i * G, G)

    @pl.when(s == 0)
    def _():
        a_hat, dinv = _adj_prep(adj_ref[...])
        ah16 = a_hat.astype(_BF16)
        ah_scr[sl] = ah16
        dinv_scr[sl] = dinv
        lin = jnp.broadcast_to(
            _lin0(gam_ref, bet_ref, w0_ref)[None], (G, N, HID))
        raw0 = _agg_raw(ah16, dinv, lin)
        _acc_stats(i, raw0, b0_ref[...], st_scr, 0)

    @pl.when(s == 1)
    def _():
        ah16 = ah_scr[sl]
        dinv = dinv_scr[sl]
        lin = jnp.broadcast_to(
            _lin0(gam_ref, bet_ref, w0_ref)[None], (G, N, HID))
        raw0 = _agg_raw(ah16, dinv, lin)
        sv, cv = _bn_coeffs(st_scr, 0, b0_ref[...], g0_ref[...], bt0_ref[...])
        h1 = jnp.maximum(raw0 * sv[None] + cv[None], 0.0)
        h1_scr[sl] = h1.astype(_BF16)
        raw1 = _agg_raw(ah16, dinv, _matmul(h1, w1_ref))
        _acc_stats(i, raw1, b1_ref[...], st_scr, 2)

    @pl.when(s == 2)
    def _():
        ah16 = ah_scr[sl]
        dinv = dinv_scr[sl]
        h1 = h1_scr[sl].astype(_F32)
        raw1 = _agg_raw(ah16, dinv, _matmul(h1, w1_ref))
        sv, cv = _bn_coeffs(st_scr, 2, b1_ref[...], g1_ref[...], bt1_ref[...])
        h2 = jnp.maximum(raw1 * sv[None] + cv[None] + h1, 0.0)
        h2_scr[sl] = h2.astype(_BF16)
        raw2 = _agg_raw(ah16, dinv, _matmul(h2, w2_ref))
        _acc_stats(i, raw2, b2_ref[...], st_scr, 4)

    @pl.when(s == 3)
    def _():
        ah16 = ah_scr[sl]
        dinv = dinv_scr[sl]
        h2 = h2_scr[sl].astype(_F32)
        raw2 = _agg_raw(ah16, dinv, _matmul(h2, w2_ref))
        sv, cv = _bn_coeffs(st_scr, 4, b2_ref[...], g2_ref[...], bt2_ref[...])
        h3 = jnp.maximum(raw2 * sv[None] + cv[None] + h2, 0.0)
        pooled = jnp.mean(h3, axis=1)
        mu = jnp.mean(pooled, axis=-1, keepdims=True)
        var = jnp.mean(pooled * pooled, axis=-1, keepdims=True) - mu * mu
        out_ref[...] = (pooled - mu) * jax.lax.rsqrt(var + EPS) \
            * lng_ref[...] + lnb_ref[...]


_FIX2 = lambda s, i: (0, 0)
_ROW_SPEC = pl.BlockSpec((1, HID), _FIX2)
_W_SPEC = pl.BlockSpec((HID, HID), _FIX2)


def kernel(adjacency_matrices, in_gamma, in_beta, W0, b0, bn_g0, bn_b0,
           W1, b1, bn_g1, bn_b1, W2, b2, bn_g2, bn_b2, ln_g, ln_b):
    adj = adjacency_matrices.astype(jnp.int8)
    gamc = in_gamma.reshape(1, 3)
    betc = in_beta.reshape(1, 3)
    row = lambda v: v.reshape(1, HID)
    g3spec = pl.BlockSpec((1, 3), _FIX2)
    w0spec = pl.BlockSpec((3, HID), _FIX2)
    adj_spec = pl.BlockSpec(
        (G, N, N), lambda s, i: (jnp.where(s == 0, i, 0), 0, 0))

    out = pl.pallas_call(
        _fused,
        grid=(4, STEPS),
        in_specs=[adj_spec, g3spec, g3spec, w0spec, _ROW_SPEC,
                  _ROW_SPEC, _ROW_SPEC, _W_SPEC, _ROW_SPEC,
                  _ROW_SPEC, _ROW_SPEC, _W_SPEC, _ROW_SPEC,
                  _ROW_SPEC, _ROW_SPEC, _ROW_SPEC, _ROW_SPEC],
        out_specs=pl.BlockSpec((G, HID), lambda s, i: (i, 0)),
        out_shape=jax.ShapeDtypeStruct((B, HID), _F32),
        scratch_shapes=[
            pltpu.VMEM((B, N, N), _BF16),
            pltpu.VMEM((B, N), _F32),
            pltpu.VMEM((B, N, HID), _BF16),
            pltpu.VMEM((B, N, HID), _BF16),
            pltpu.VMEM((8, HID), _F32),
        ],
    )(adj, gamc, betc, W0, row(b0), row(bn_g0), row(bn_b0),
      W1, row(b1), row(bn_g1), row(bn_b1),
      W2, row(b2), row(bn_g2), row(bn_b2), row(ln_g), row(ln_b))

    return out
